# Initial kernel scaffold; baseline (speedup 1.0000x reference)
import jax
import jax.numpy as jnp
from jax.experimental import pallas as pl


def kernel(z, W, emb):
    raise NotImplementedError("write your pallas kernel here")


# speedup vs baseline: 3.8695x; 3.8695x over previous
import jax
import jax.numpy as jnp
from jax.experimental import pallas as pl
from jax.experimental.pallas import tpu as pltpu

_B, _C_IN, _N = 8, 256, 196
_D, _K = 64, 1024


def _vq_body(z_ref, w_ref, emb_ref, out_ref):
    z = z_ref[0]
    w = w_ref[...]
    emb = emb_ref[...]

    hi = jax.lax.Precision.HIGHEST
    ze = jnp.dot(w.astype(jnp.bfloat16), z.astype(jnp.bfloat16),
                 preferred_element_type=jnp.float32)
    e_sq = jnp.sum(emb * emb, axis=1, keepdims=True)
    s = e_sq - 2.0 * jnp.dot(emb, ze, preferred_element_type=jnp.float32,
                             precision=hi)

    m = jnp.min(s, axis=0, keepdims=True)
    kio = jax.lax.broadcasted_iota(jnp.int32, (_K, _N), 0)
    idx = jnp.min(jnp.where(s <= m, kio, _K), axis=0)
    onehot = (kio == idx[None, :]).astype(jnp.float32)
    zq = jnp.dot(emb.T, onehot, preferred_element_type=jnp.float32,
                 precision=hi)
    out_ref[0] = zq


def kernel(z, W, emb):
    return pl.pallas_call(
        _vq_body,
        grid=(_B,),
        in_specs=[
            pl.BlockSpec((1, _C_IN, _N), lambda b: (b, 0, 0)),
            pl.BlockSpec((_D, _C_IN), lambda b: (0, 0)),
            pl.BlockSpec((_K, _D), lambda b: (0, 0)),
        ],
        out_specs=pl.BlockSpec((1, _D, _N), lambda b: (b, 0, 0)),
        out_shape=jax.ShapeDtypeStruct((_B, _D, _N), jnp.float32),
    )(z, W, emb)


# speedup vs baseline: 6.5695x; 1.6977x over previous
import jax
import jax.numpy as jnp
from jax.experimental import pallas as pl
from jax.experimental.pallas import tpu as pltpu

_B, _C_IN, _N = 8, 256, 196
_D, _K = 64, 1024
_BN = _B * _N


def _vq_body(z_ref, w_ref, emb_ref, out_ref):
    w = w_ref[...]
    emb = emb_ref[...]
    hi = jax.lax.Precision.HIGHEST

    wb = w.astype(jnp.bfloat16)
    ze = jnp.concatenate(
        [jnp.dot(wb, z_ref[b].astype(jnp.bfloat16),
                 preferred_element_type=jnp.float32) for b in range(_B)],
        axis=1)

    e_sq = jnp.sum(emb * emb, axis=1, keepdims=True)
    s = e_sq - 2.0 * jnp.dot(emb, ze, preferred_element_type=jnp.float32,
                             precision=hi)

    m = jnp.min(s, axis=0, keepdims=True)
    kio = jax.lax.broadcasted_iota(jnp.int32, (_K, _BN), 0)
    idx = jnp.min(jnp.where(s <= m, kio, _K), axis=0)
    onehot = (kio == idx[None, :]).astype(jnp.float32)
    zq = jnp.dot(emb.T, onehot, preferred_element_type=jnp.float32,
                 precision=hi)
    for b in range(_B):
        out_ref[b] = zq[:, b * _N:(b + 1) * _N]


def kernel(z, W, emb):
    return pl.pallas_call(
        _vq_body,
        in_specs=[
            pl.BlockSpec(memory_space=pltpu.VMEM),
            pl.BlockSpec(memory_space=pltpu.VMEM),
            pl.BlockSpec(memory_space=pltpu.VMEM),
        ],
        out_specs=pl.BlockSpec(memory_space=pltpu.VMEM),
        out_shape=jax.ShapeDtypeStruct((_B, _D, _N), jnp.float32),
    )(z, W, emb)


# speedup vs baseline: 8.3958x; 1.2780x over previous
import jax
import jax.numpy as jnp
from jax.experimental import pallas as pl
from jax.experimental.pallas import tpu as pltpu

_B, _C_IN, _N = 8, 256, 196
_D, _K = 64, 1024
_BN = _B * _N


def _vq_body(z_ref, w_ref, emb_ref, out_ref):
    w = w_ref[...]
    emb = emb_ref[...]
    hi = jax.lax.Precision.HIGHEST

    wb = w.astype(jnp.bfloat16)
    ze = jnp.concatenate(
        [jnp.dot(wb, z_ref[b].astype(jnp.bfloat16),
                 preferred_element_type=jnp.float32) for b in range(_B)],
        axis=1)

    e_sq = jnp.sum(emb * emb, axis=1, keepdims=True)
    eh = emb.astype(jnp.bfloat16)
    el = (emb - eh.astype(jnp.float32)).astype(jnp.bfloat16)
    zh = ze.astype(jnp.bfloat16)
    zl = (ze - zh.astype(jnp.float32)).astype(jnp.bfloat16)
    dot3 = (jnp.dot(eh, zh, preferred_element_type=jnp.float32) +
            jnp.dot(eh, zl, preferred_element_type=jnp.float32) +
            jnp.dot(el, zh, preferred_element_type=jnp.float32))
    s = e_sq - 2.0 * dot3

    m = jnp.min(s, axis=0, keepdims=True)
    kio = jax.lax.broadcasted_iota(jnp.int32, (_K, _BN), 0)
    idx = jnp.min(jnp.where(s <= m, kio, _K), axis=0)
    onehot = (kio == idx[None, :]).astype(jnp.bfloat16)
    zq = (jnp.dot(eh.T, onehot, preferred_element_type=jnp.float32) +
          jnp.dot(el.T, onehot, preferred_element_type=jnp.float32))
    for b in range(_B):
        out_ref[b] = zq[:, b * _N:(b + 1) * _N]


def kernel(z, W, emb):
    return pl.pallas_call(
        _vq_body,
        in_specs=[
            pl.BlockSpec(memory_space=pltpu.VMEM),
            pl.BlockSpec(memory_space=pltpu.VMEM),
            pl.BlockSpec(memory_space=pltpu.VMEM),
        ],
        out_specs=pl.BlockSpec(memory_space=pltpu.VMEM),
        out_shape=jax.ShapeDtypeStruct((_B, _D, _N), jnp.float32),
    )(z, W, emb)
